# Initial kernel scaffold; baseline (speedup 1.0000x reference)
import jax
import jax.numpy as jnp
from jax.experimental import pallas as pl


def kernel(x_node, x_edgenode, x_cyclenode, src0, dst0, src1, dst1, src2, dst2, src3, dst3, src4, dst4, Wn, bn_, We, be, Wc, bc, fc1_w, fc1_b, al1, ar1, fc2_w, fc2_b, al2, ar2, g1, beta1, g2, beta2, Wout, bout):
    raise NotImplementedError("write your pallas kernel here")


# speedup vs baseline: 49.4610x; 49.4610x over previous
import functools

import jax
import jax.numpy as jnp
from jax import lax
from jax.experimental import pallas as pl
from jax.experimental.pallas import tpu as pltpu
from jax.experimental.pallas import tpu_sc as plsc

N_NODE = 100000
N_EDGE = 50000
N_CYC = 10000
H, D, HD = 2, 16, 32
PW = 24
EW = 16
AW = 20


def _cdiv(a, b):
    return -(-a // b)


def _chunk(n_dst):
    return _cdiv(_cdiv(n_dst // 2 + 1, 16), 8) * 8


@functools.lru_cache(maxsize=None)
def _edge_sc(n_dst, e_pad, B):
    nh = n_dst // 2
    chunk = _chunk(n_dst)
    rows = 16 * chunk
    NB = e_pad // (16 * B)
    C = B // 128
    G = B // 16
    mesh = plsc.VectorSubcoreMesh(core_axis_name="c", subcore_axis_name="s",
                                  num_cores=2, num_subcores=16)

    @functools.partial(
        pl.kernel,
        out_type=jax.ShapeDtypeStruct((H, 2, rows, AW), jnp.float32),
        mesh=mesh,
        scratch_types=[
            pltpu.VMEM((B,), jnp.int32),
            pltpu.VMEM((B,), jnp.int32),
            pltpu.VMEM((C, 128), jnp.int32),
            pltpu.VMEM((B, PW), jnp.float32),
            pltpu.VMEM((B, EW), jnp.float32),
            pltpu.VMEM((B, AW), jnp.float32),
            pltpu.VMEM_SHARED((rows, AW), jnp.float32),
            pltpu.SemaphoreType.DMA,
        ],
        compiler_params=pltpu.CompilerParams(needs_layout_passes=False,
                                             use_tc_tiling_on_sc=False),
    )
    def k(src_h, dst_h, p0_h, p1_h, er_h, zrow_h, out_h,
          sidx, didx, lidx, rowsv, ersv, sbuf, acc, sem):
        c = lax.axis_index("c")
        s = lax.axis_index("s")
        for h in range(H):
            p_h = (p0_h, p1_h)[h]
            pltpu.sync_copy(zrow_h, acc.at[pl.ds(s * chunk, chunk)])
            plsc.subcore_barrier()

            def blk(b, carry):
                base = (s * NB + b) * B
                pltpu.sync_copy(src_h.at[pl.ds(base, B)], sidx)
                pltpu.sync_copy(dst_h.at[pl.ds(base, B)], didx)

                def fire(j, cr):
                    pltpu.async_copy(p_h.at[sidx.at[pl.ds(j * 128, 128)]],
                                     rowsv.at[pl.ds(j * 128, 128)], sem)
                    pltpu.async_copy(er_h.at[didx.at[pl.ds(j * 128, 128)]],
                                     ersv.at[pl.ds(j * 128, 128)], sem)
                    return cr
                lax.fori_loop(0, C, fire, 0)

                def drain(j, cr):
                    pltpu.make_async_copy(
                        p_h.at[sidx.at[pl.ds(j * 128, 128)]],
                        rowsv.at[pl.ds(j * 128, 128)], sem).wait()
                    pltpu.make_async_copy(
                        er_h.at[didx.at[pl.ds(j * 128, 128)]],
                        ersv.at[pl.ds(j * 128, 128)], sem).wait()
                    return cr
                lax.fori_loop(0, C, drain, 0)

                def grp(j, cr):
                    eb = j * 16
                    eidx = eb + lax.iota(jnp.int32, 16)
                    z16 = jnp.zeros((16,), jnp.int32)
                    el = plsc.load_gather(rowsv, [eidx, z16])
                    er = plsc.load_gather(ersv, [eidx, z16 + h])
                    e = el + er
                    w = jnp.exp(jnp.where(e > 0, e, e * jnp.float32(0.2)))
                    d16 = didx[pl.ds(eb, 16)]
                    loc = d16 - c * nh
                    okm = (loc >= 0) & (loc < nh)
                    loce = jnp.where(okm, loc, nh)
                    lidx[j // 8, pl.ds((j % 8) * 16, 16)] = loce
                    plsc.store_scatter(sbuf, [eidx, z16], w)
                    for k2 in range(16):
                        f = plsc.load_gather(rowsv, [eidx, z16 + (8 + k2)])
                        plsc.store_scatter(sbuf, [eidx, z16 + (4 + k2)], f * w)
                    return cr
                lax.fori_loop(0, G, grp, 0)

                def scat(j, cr):
                    pltpu.sync_copy(sbuf.at[pl.ds(j * 128, 128)],
                                    acc.at[lidx.at[j]], add=True)
                    return cr
                lax.fori_loop(0, C, scat, 0)
                return carry
            lax.fori_loop(0, NB, blk, 0)
            plsc.subcore_barrier()
            pltpu.sync_copy(acc.at[pl.ds(s * chunk, chunk)],
                            out_h.at[h, c, pl.ds(s * chunk, chunk)])
            plsc.subcore_barrier()

    return k


def _pack_tc(N, K, n_p, n_er, x, Wp, bp, ALm, Wer, ber, ARm):
    BR = 400
    grid = (N // BR,)

    def body(x_ref, wp_ref, bp_ref, al_ref, wer_ref, ber_ref, ar_ref, *outs):
        x_b = x_ref[...]
        for p in range(n_p):
            fs = jnp.dot(x_b, wp_ref[p], preferred_element_type=jnp.float32) + bp_ref[p]
            el = jnp.dot(fs, al_ref[p], preferred_element_type=jnp.float32)
            for h in range(H):
                outs[2 * p + h][...] = jnp.concatenate(
                    [el[:, h:h + 1], jnp.zeros((BR, 7), jnp.float32),
                     fs[:, 16 * h:16 * (h + 1)]], axis=1)
        for q in range(n_er):
            fd = jnp.dot(x_b, wer_ref[q], preferred_element_type=jnp.float32) + ber_ref[q]
            er = jnp.dot(fd, ar_ref[q], preferred_element_type=jnp.float32)
            outs[2 * n_p + q][...] = jnp.concatenate(
                [er, jnp.zeros((BR, 14), jnp.float32)], axis=1)

    full = lambda shape: pl.BlockSpec(shape, lambda j: tuple(0 for _ in shape))
    out_shapes = ([jax.ShapeDtypeStruct((N, PW), jnp.float32)] * (2 * n_p)
                  + [jax.ShapeDtypeStruct((N, EW), jnp.float32)] * n_er)
    out_specs = ([pl.BlockSpec((BR, PW), lambda j: (j, 0))] * (2 * n_p)
                 + [pl.BlockSpec((BR, EW), lambda j: (j, 0))] * n_er)
    return pl.pallas_call(
        body,
        grid=grid,
        in_specs=[
            pl.BlockSpec((BR, K), lambda j: (j, 0)),
            full((n_p, K, HD)), full((n_p, HD)), full((n_p, HD, H)),
            full((n_er, K, HD)), full((n_er, HD)), full((n_er, HD, H)),
        ],
        out_specs=out_specs,
        out_shape=out_shapes,
    )(x, Wp, bp, ALm, Wer, ber, ARm)


def _combine_tc(N, n_acc, scale, accs, bsum, want_z):
    BR = 200
    nh = N // 2
    nj = nh // BR
    grid = (2, nj)
    rows = 16 * _chunk(N)

    def body(*refs):
        acc_refs = refs[:n_acc]
        bsum_ref = refs[n_acc]
        if want_z:
            z_ref, st_ref = refs[n_acc + 1], refs[n_acc + 2]
        else:
            st_ref = refs[n_acc + 1]
        q = None
        for a_ref in acc_refs:
            a = a_ref[:, 0]
            den = a[:, :, 0:1]
            num = a[:, :, 4:20]
            t = (num / (den + jnp.float32(1e-9)))
            t = jnp.concatenate([t[0], t[1]], axis=1)
            q = t if q is None else q + t
        z = jnp.maximum((q + bsum_ref[...]) * jnp.float32(scale), 0.0)
        if want_z:
            z_ref[...] = z
        first = (pl.program_id(0) == 0) & (pl.program_id(1) == 0)

        @pl.when(first)
        def _():
            st_ref[...] = jnp.zeros((2, HD), jnp.float32)
        st_ref[...] = st_ref[...] + jnp.stack([z.sum(0), (z * z).sum(0)])

    in_specs = ([pl.BlockSpec((H, 1, BR, AW), lambda hc, j: (0, hc, j, 0))] * n_acc
                + [pl.BlockSpec((HD,), lambda hc, j: (0,))])
    out_shapes = [jax.ShapeDtypeStruct((2, HD), jnp.float32)]
    out_specs = [pl.BlockSpec((2, HD), lambda hc, j: (0, 0))]
    if want_z:
        out_shapes.insert(0, jax.ShapeDtypeStruct((N, HD), jnp.float32))
        out_specs.insert(0, pl.BlockSpec((BR, HD), lambda hc, j: (hc * nj + j, 0)))
    return pl.pallas_call(
        body, grid=grid, in_specs=in_specs,
        out_specs=out_specs, out_shape=out_shapes,
    )(*accs, bsum)


def _readout_tc(st_n, st_e, st_c, g2, beta2, Wout, bout):
    def body(sn, se, sc, g_ref, b_ref, w_ref, bo_ref, o_ref):
        parts = []
        for st_ref, n in ((sn, N_NODE), (se, N_EDGE), (sc, N_CYC)):
            st = st_ref[...]
            mean = st[0] * jnp.float32(1.0 / n)
            var = st[1] * jnp.float32(1.0 / n) - mean * mean
            inv = lax.rsqrt(var + jnp.float32(1e-5))
            parts.append((mean - mean) * inv * g_ref[...] + b_ref[...])
        pooled = jnp.concatenate(parts)
        o_ref[...] = (jnp.sum(pooled * w_ref[..., 0]) + bo_ref[0]).reshape(1, 1)

    full = lambda shape: pl.BlockSpec(shape, lambda: tuple(0 for _ in shape))
    return pl.pallas_call(
        body,
        in_specs=[full((2, HD))] * 3 + [full((HD,)), full((HD,)),
                                        full((3 * HD, 1)), full((1,))],
        out_specs=pl.BlockSpec((1, 1), lambda: (0, 0)),
        out_shape=jax.ShapeDtypeStruct((1, 1), jnp.float32),
    )(st_n, st_e, st_c, g2, beta2, Wout, bout)


def _almat(a):
    M = jnp.zeros((HD, H), jnp.float32)
    M = M.at[0:16, 0].set(a[0])
    M = M.at[16:32, 1].set(a[1])
    return M


def _pad_edges(src, dst, n_dst, B):
    e = src.shape[0]
    e_pad = _cdiv(e, 16 * B) * 16 * B
    pad = e_pad - e
    src = jnp.concatenate([src.astype(jnp.int32), jnp.zeros((pad,), jnp.int32)])
    dst = jnp.concatenate([dst.astype(jnp.int32),
                           jnp.full((pad,), n_dst, jnp.int32)])
    return src, dst, e_pad


def _pad_er(er):
    return jnp.concatenate([er, jnp.zeros((16, EW), jnp.float32)], axis=0)


_NDST = (N_NODE, N_EDGE, N_NODE, N_CYC, N_NODE)
_B = 512


def kernel(x_node, x_edgenode, x_cyclenode, src0, dst0, src1, dst1, src2, dst2,
           src3, dst3, src4, dst4, Wn, bn_, We, be, Wc, bc, fc1_w, fc1_b, al1,
           ar1, fc2_w, fc2_b, al2, ar2, g1, beta1, g2, beta2, Wout, bout):
    srcs = (src0, src1, src2, src3, src4)
    dsts = (dst0, dst1, dst2, dst3, dst4)
    ep = [_pad_edges(srcs[r], dsts[r], _NDST[r], _B) for r in range(5)]
    zrow = {n: jnp.zeros((_chunk(n), AW), jnp.float32)
            for n in (N_NODE, N_EDGE, N_CYC)}

    def layer(xs, K_n, Wbase, bbase, al, ar, fcw, fcb):
        ALs = [_almat(al[r]) for r in range(5)]
        ARs = [_almat(ar[r]) for r in range(5)]
        W = [Wbase[t] @ fcw[r] for t, r in
             ((0, 0), (0, 1), (1, 2), (1, 3), (2, 4))]
        bvec = [bbase[t] @ fcw[r] for t, r in
                ((0, 0), (0, 1), (1, 2), (1, 3), (2, 4))]
        Wd = [Wbase[t] @ fcw[r] for t, r in
              ((0, 0), (1, 1), (0, 2), (2, 3), (0, 4))]
        bd = [bbase[t] @ fcw[r] for t, r in
              ((0, 0), (1, 1), (0, 2), (2, 3), (0, 4))]
        K = [K_n[0], K_n[0], K_n[1], K_n[1], K_n[2]]
        P0a, P0b, P1a, P1b, er0, er2, er4 = _pack_tc(
            N_NODE, K[0], 2, 3, xs[0],
            jnp.stack([W[0], W[1]]), jnp.stack([bvec[0], bvec[1]]),
            jnp.stack([ALs[0], ALs[1]]),
            jnp.stack([Wd[0], Wd[2], Wd[4]]), jnp.stack([bd[0], bd[2], bd[4]]),
            jnp.stack([ARs[0], ARs[2], ARs[4]]))
        P2a, P2b, P3a, P3b, er1 = _pack_tc(
            N_EDGE, K[2], 2, 1, xs[1],
            jnp.stack([W[2], W[3]]), jnp.stack([bvec[2], bvec[3]]),
            jnp.stack([ALs[2], ALs[3]]),
            jnp.stack([Wd[1]]), jnp.stack([bd[1]]), jnp.stack([ARs[1]]))
        P4a, P4b, er3 = _pack_tc(
            N_CYC, K[4], 1, 1, xs[2],
            jnp.stack([W[4]]), jnp.stack([bvec[4]]), jnp.stack([ALs[4]]),
            jnp.stack([Wd[3]]), jnp.stack([bd[3]]), jnp.stack([ARs[3]]))
        P = ((P0a, P0b), (P1a, P1b), (P2a, P2b), (P3a, P3b), (P4a, P4b))
        er = tuple(_pad_er(t) for t in (er0, er1, er2, er3, er4))
        accs = []
        for r in range(5):
            srcp, dstp, e_pad = ep[r]
            accs.append(_edge_sc(_NDST[r], e_pad, _B)(
                srcp, dstp, P[r][0], P[r][1], er[r], zrow[_NDST[r]]))
        return accs

    accs1 = layer((x_node, x_edgenode, x_cyclenode), (128, 16, 8),
                  (Wn, We, Wc), (bn_, be, bc), al1, ar1, fc1_w, fc1_b)
    z_n, st_n = _combine_tc(N_NODE, 3, 1.0 / 3.0,
                            (accs1[0], accs1[2], accs1[4]),
                            fc1_b[0] + fc1_b[2] + fc1_b[4], True)
    z_e, st_e = _combine_tc(N_EDGE, 1, 1.0, (accs1[1],), fc1_b[1], True)
    z_c, st_c = _combine_tc(N_CYC, 1, 1.0, (accs1[3],), fc1_b[3], True)

    def fold(st, n):
        mean = st[0] / n
        var = st[1] / n - mean * mean
        s = g1 * lax.rsqrt(var + 1e-5)
        return jnp.diag(s), beta1 - mean * s
    Sn, on = fold(st_n, N_NODE)
    Se, oe = fold(st_e, N_EDGE)
    Sc, oc = fold(st_c, N_CYC)

    accs2 = layer((z_n, z_e, z_c), (HD, HD, HD),
                  (Sn, Se, Sc), (on, oe, oc), al2, ar2, fc2_w, fc2_b)
    st_n2 = _combine_tc(N_NODE, 3, 1.0 / 3.0,
                        (accs2[0], accs2[2], accs2[4]),
                        fc2_b[0] + fc2_b[2] + fc2_b[4], False)[0]
    st_e2 = _combine_tc(N_EDGE, 1, 1.0, (accs2[1],), fc2_b[1], False)[0]
    st_c2 = _combine_tc(N_CYC, 1, 1.0, (accs2[3],), fc2_b[3], False)[0]

    return _readout_tc(st_n2, st_e2, st_c2, g2, beta2, Wout, bout)


# speedup vs baseline: 49.6677x; 1.0042x over previous
import functools

import jax
import jax.numpy as jnp
from jax import lax
from jax.experimental import pallas as pl
from jax.experimental.pallas import tpu as pltpu
from jax.experimental.pallas import tpu_sc as plsc

N_NODE = 100000
N_EDGE = 50000
N_CYC = 10000
H, D, HD = 2, 16, 32
PW = 24
EW = 16
AW = 20


def _cdiv(a, b):
    return -(-a // b)


def _chunk(n_dst):
    return _cdiv(_cdiv(n_dst // 2 + 1, 16), 8) * 8


@functools.lru_cache(maxsize=None)
def _edge_sc(n_dst, e_pad, B):
    nh = n_dst // 2
    chunk = _chunk(n_dst)
    rows = 16 * chunk
    NB = e_pad // (16 * B)
    C = B // 128
    G = B // 16
    mesh = plsc.VectorSubcoreMesh(core_axis_name="c", subcore_axis_name="s",
                                  num_cores=2, num_subcores=16)

    @functools.partial(
        pl.kernel,
        out_type=jax.ShapeDtypeStruct((H, 2, rows, AW), jnp.float32),
        mesh=mesh,
        scratch_types=[
            pltpu.VMEM((B,), jnp.int32),
            pltpu.VMEM((B,), jnp.int32),
            pltpu.VMEM((C, 128), jnp.int32),
            pltpu.VMEM((B, PW), jnp.float32),
            pltpu.VMEM((B, EW), jnp.float32),
            pltpu.VMEM((B, AW), jnp.float32),
            pltpu.VMEM_SHARED((rows, AW), jnp.float32),
            pltpu.SemaphoreType.DMA,
        ],
        compiler_params=pltpu.CompilerParams(needs_layout_passes=False,
                                             use_tc_tiling_on_sc=False),
    )
    def k(src_h, dst_h, p0_h, p1_h, er_h, zrow_h, out_h,
          sidx, didx, lidx, rowsv, ersv, sbuf, acc, sem):
        c = lax.axis_index("c")
        s = lax.axis_index("s")
        for h in range(H):
            p_h = (p0_h, p1_h)[h]
            pltpu.sync_copy(zrow_h, acc.at[pl.ds(s * chunk, chunk)])
            plsc.subcore_barrier()

            def blk(b, carry):
                base = (s * NB + b) * B
                pltpu.async_copy(src_h.at[pl.ds(base, B)], sidx, sem)
                pltpu.async_copy(dst_h.at[pl.ds(base, B)], didx, sem)
                pltpu.make_async_copy(src_h.at[pl.ds(base, B)], sidx, sem).wait()
                pltpu.make_async_copy(dst_h.at[pl.ds(base, B)], didx, sem).wait()

                def fire(j, cr):
                    pltpu.async_copy(p_h.at[sidx.at[pl.ds(j * 128, 128)]],
                                     rowsv.at[pl.ds(j * 128, 128)], sem)
                    pltpu.async_copy(er_h.at[didx.at[pl.ds(j * 128, 128)]],
                                     ersv.at[pl.ds(j * 128, 128)], sem)
                    return cr
                lax.fori_loop(0, C, fire, 0)

                def drain(j, cr):
                    pltpu.make_async_copy(
                        p_h.at[sidx.at[pl.ds(j * 128, 128)]],
                        rowsv.at[pl.ds(j * 128, 128)], sem).wait()
                    pltpu.make_async_copy(
                        er_h.at[didx.at[pl.ds(j * 128, 128)]],
                        ersv.at[pl.ds(j * 128, 128)], sem).wait()
                    return cr
                lax.fori_loop(0, C, drain, 0)

                def grp(j, cr):
                    eb = j * 16
                    eidx = eb + lax.iota(jnp.int32, 16)
                    z16 = jnp.zeros((16,), jnp.int32)
                    el = plsc.load_gather(rowsv, [eidx, z16])
                    er = plsc.load_gather(ersv, [eidx, z16 + h])
                    e = el + er
                    w = jnp.exp(jnp.where(e > 0, e, e * jnp.float32(0.2)))
                    d16 = didx[pl.ds(eb, 16)]
                    loc = d16 - c * nh
                    okm = (loc >= 0) & (loc < nh)
                    loce = jnp.where(okm, loc, nh)
                    lidx[j // 8, pl.ds((j % 8) * 16, 16)] = loce
                    plsc.store_scatter(sbuf, [eidx, z16], w)
                    for k2 in range(16):
                        f = plsc.load_gather(rowsv, [eidx, z16 + (8 + k2)])
                        plsc.store_scatter(sbuf, [eidx, z16 + (4 + k2)], f * w)
                    return cr
                lax.fori_loop(0, G, grp, 0)

                def scat(j, cr):
                    pltpu.async_copy(sbuf.at[pl.ds(j * 128, 128)],
                                     acc.at[lidx.at[j]], sem, add=True)
                    return cr
                lax.fori_loop(0, C, scat, 0)

                def scat_drain(j, cr):
                    pltpu.make_async_copy(sbuf.at[pl.ds(j * 128, 128)],
                                          acc.at[lidx.at[j]], sem).wait()
                    return cr
                lax.fori_loop(0, C, scat_drain, 0)
                return carry
            lax.fori_loop(0, NB, blk, 0)
            plsc.subcore_barrier()
            pltpu.sync_copy(acc.at[pl.ds(s * chunk, chunk)],
                            out_h.at[h, c, pl.ds(s * chunk, chunk)])
            plsc.subcore_barrier()

    return k


def _pack_tc(N, K, n_p, n_er, x, Wp, bp, ALm, Wer, ber, ARm):
    BR = 400
    grid = (N // BR,)

    def body(x_ref, wp_ref, bp_ref, al_ref, wer_ref, ber_ref, ar_ref, *outs):
        x_b = x_ref[...]
        for p in range(n_p):
            fs = jnp.dot(x_b, wp_ref[p], preferred_element_type=jnp.float32) + bp_ref[p]
            el = jnp.dot(fs, al_ref[p], preferred_element_type=jnp.float32)
            for h in range(H):
                outs[2 * p + h][...] = jnp.concatenate(
                    [el[:, h:h + 1], jnp.zeros((BR, 7), jnp.float32),
                     fs[:, 16 * h:16 * (h + 1)]], axis=1)
        for q in range(n_er):
            fd = jnp.dot(x_b, wer_ref[q], preferred_element_type=jnp.float32) + ber_ref[q]
            er = jnp.dot(fd, ar_ref[q], preferred_element_type=jnp.float32)
            outs[2 * n_p + q][...] = jnp.concatenate(
                [er, jnp.zeros((BR, 14), jnp.float32)], axis=1)

    full = lambda shape: pl.BlockSpec(shape, lambda j: tuple(0 for _ in shape))
    out_shapes = ([jax.ShapeDtypeStruct((N, PW), jnp.float32)] * (2 * n_p)
                  + [jax.ShapeDtypeStruct((N, EW), jnp.float32)] * n_er)
    out_specs = ([pl.BlockSpec((BR, PW), lambda j: (j, 0))] * (2 * n_p)
                 + [pl.BlockSpec((BR, EW), lambda j: (j, 0))] * n_er)
    return pl.pallas_call(
        body,
        grid=grid,
        in_specs=[
            pl.BlockSpec((BR, K), lambda j: (j, 0)),
            full((n_p, K, HD)), full((n_p, HD)), full((n_p, HD, H)),
            full((n_er, K, HD)), full((n_er, HD)), full((n_er, HD, H)),
        ],
        out_specs=out_specs,
        out_shape=out_shapes,
    )(x, Wp, bp, ALm, Wer, ber, ARm)


def _combine_tc(N, n_acc, scale, accs, bsum, want_z):
    BR = 200
    nh = N // 2
    nj = nh // BR
    grid = (2, nj)
    rows = 16 * _chunk(N)

    def body(*refs):
        acc_refs = refs[:n_acc]
        bsum_ref = refs[n_acc]
        if want_z:
            z_ref, st_ref = refs[n_acc + 1], refs[n_acc + 2]
        else:
            st_ref = refs[n_acc + 1]
        q = None
        for a_ref in acc_refs:
            a = a_ref[:, 0]
            den = a[:, :, 0:1]
            num = a[:, :, 4:20]
            t = (num / (den + jnp.float32(1e-9)))
            t = jnp.concatenate([t[0], t[1]], axis=1)
            q = t if q is None else q + t
        z = jnp.maximum((q + bsum_ref[...]) * jnp.float32(scale), 0.0)
        if want_z:
            z_ref[...] = z
        first = (pl.program_id(0) == 0) & (pl.program_id(1) == 0)

        @pl.when(first)
        def _():
            st_ref[...] = jnp.zeros((2, HD), jnp.float32)
        st_ref[...] = st_ref[...] + jnp.stack([z.sum(0), (z * z).sum(0)])

    in_specs = ([pl.BlockSpec((H, 1, BR, AW), lambda hc, j: (0, hc, j, 0))] * n_acc
                + [pl.BlockSpec((HD,), lambda hc, j: (0,))])
    out_shapes = [jax.ShapeDtypeStruct((2, HD), jnp.float32)]
    out_specs = [pl.BlockSpec((2, HD), lambda hc, j: (0, 0))]
    if want_z:
        out_shapes.insert(0, jax.ShapeDtypeStruct((N, HD), jnp.float32))
        out_specs.insert(0, pl.BlockSpec((BR, HD), lambda hc, j: (hc * nj + j, 0)))
    return pl.pallas_call(
        body, grid=grid, in_specs=in_specs,
        out_specs=out_specs, out_shape=out_shapes,
    )(*accs, bsum)


def _readout_tc(st_n, st_e, st_c, g2, beta2, Wout, bout):
    def body(sn, se, sc, g_ref, b_ref, w_ref, bo_ref, o_ref):
        parts = []
        for st_ref, n in ((sn, N_NODE), (se, N_EDGE), (sc, N_CYC)):
            st = st_ref[...]
            mean = st[0] * jnp.float32(1.0 / n)
            var = st[1] * jnp.float32(1.0 / n) - mean * mean
            inv = lax.rsqrt(var + jnp.float32(1e-5))
            parts.append((mean - mean) * inv * g_ref[...] + b_ref[...])
        pooled = jnp.concatenate(parts)
        o_ref[...] = (jnp.sum(pooled * w_ref[..., 0]) + bo_ref[0]).reshape(1, 1)

    full = lambda shape: pl.BlockSpec(shape, lambda: tuple(0 for _ in shape))
    return pl.pallas_call(
        body,
        in_specs=[full((2, HD))] * 3 + [full((HD,)), full((HD,)),
                                        full((3 * HD, 1)), full((1,))],
        out_specs=pl.BlockSpec((1, 1), lambda: (0, 0)),
        out_shape=jax.ShapeDtypeStruct((1, 1), jnp.float32),
    )(st_n, st_e, st_c, g2, beta2, Wout, bout)


def _almat(a):
    M = jnp.zeros((HD, H), jnp.float32)
    M = M.at[0:16, 0].set(a[0])
    M = M.at[16:32, 1].set(a[1])
    return M


def _pad_edges(src, dst, n_dst, B):
    e = src.shape[0]
    e_pad = _cdiv(e, 16 * B) * 16 * B
    pad = e_pad - e
    src = jnp.concatenate([src.astype(jnp.int32), jnp.zeros((pad,), jnp.int32)])
    dst = jnp.concatenate([dst.astype(jnp.int32),
                           jnp.full((pad,), n_dst, jnp.int32)])
    return src, dst, e_pad


def _pad_er(er):
    return jnp.concatenate([er, jnp.zeros((16, EW), jnp.float32)], axis=0)


_NDST = (N_NODE, N_EDGE, N_NODE, N_CYC, N_NODE)
_B = 768


def kernel(x_node, x_edgenode, x_cyclenode, src0, dst0, src1, dst1, src2, dst2,
           src3, dst3, src4, dst4, Wn, bn_, We, be, Wc, bc, fc1_w, fc1_b, al1,
           ar1, fc2_w, fc2_b, al2, ar2, g1, beta1, g2, beta2, Wout, bout):
    srcs = (src0, src1, src2, src3, src4)
    dsts = (dst0, dst1, dst2, dst3, dst4)
    ep = [_pad_edges(srcs[r], dsts[r], _NDST[r], _B) for r in range(5)]
    zrow = {n: jnp.zeros((_chunk(n), AW), jnp.float32)
            for n in (N_NODE, N_EDGE, N_CYC)}

    def layer(xs, K_n, Wbase, bbase, al, ar, fcw, fcb):
        ALs = [_almat(al[r]) for r in range(5)]
        ARs = [_almat(ar[r]) for r in range(5)]
        W = [Wbase[t] @ fcw[r] for t, r in
             ((0, 0), (0, 1), (1, 2), (1, 3), (2, 4))]
        bvec = [bbase[t] @ fcw[r] for t, r in
                ((0, 0), (0, 1), (1, 2), (1, 3), (2, 4))]
        Wd = [Wbase[t] @ fcw[r] for t, r in
              ((0, 0), (1, 1), (0, 2), (2, 3), (0, 4))]
        bd = [bbase[t] @ fcw[r] for t, r in
              ((0, 0), (1, 1), (0, 2), (2, 3), (0, 4))]
        K = [K_n[0], K_n[0], K_n[1], K_n[1], K_n[2]]
        P0a, P0b, P1a, P1b, er0, er2, er4 = _pack_tc(
            N_NODE, K[0], 2, 3, xs[0],
            jnp.stack([W[0], W[1]]), jnp.stack([bvec[0], bvec[1]]),
            jnp.stack([ALs[0], ALs[1]]),
            jnp.stack([Wd[0], Wd[2], Wd[4]]), jnp.stack([bd[0], bd[2], bd[4]]),
            jnp.stack([ARs[0], ARs[2], ARs[4]]))
        P2a, P2b, P3a, P3b, er1 = _pack_tc(
            N_EDGE, K[2], 2, 1, xs[1],
            jnp.stack([W[2], W[3]]), jnp.stack([bvec[2], bvec[3]]),
            jnp.stack([ALs[2], ALs[3]]),
            jnp.stack([Wd[1]]), jnp.stack([bd[1]]), jnp.stack([ARs[1]]))
        P4a, P4b, er3 = _pack_tc(
            N_CYC, K[4], 1, 1, xs[2],
            jnp.stack([W[4]]), jnp.stack([bvec[4]]), jnp.stack([ALs[4]]),
            jnp.stack([Wd[3]]), jnp.stack([bd[3]]), jnp.stack([ARs[3]]))
        P = ((P0a, P0b), (P1a, P1b), (P2a, P2b), (P3a, P3b), (P4a, P4b))
        er = tuple(_pad_er(t) for t in (er0, er1, er2, er3, er4))
        accs = []
        for r in range(5):
            srcp, dstp, e_pad = ep[r]
            accs.append(_edge_sc(_NDST[r], e_pad, _B)(
                srcp, dstp, P[r][0], P[r][1], er[r], zrow[_NDST[r]]))
        return accs

    accs1 = layer((x_node, x_edgenode, x_cyclenode), (128, 16, 8),
                  (Wn, We, Wc), (bn_, be, bc), al1, ar1, fc1_w, fc1_b)
    z_n, st_n = _combine_tc(N_NODE, 3, 1.0 / 3.0,
                            (accs1[0], accs1[2], accs1[4]),
                            fc1_b[0] + fc1_b[2] + fc1_b[4], True)
    z_e, st_e = _combine_tc(N_EDGE, 1, 1.0, (accs1[1],), fc1_b[1], True)
    z_c, st_c = _combine_tc(N_CYC, 1, 1.0, (accs1[3],), fc1_b[3], True)

    def fold(st, n):
        mean = st[0] / n
        var = st[1] / n - mean * mean
        s = g1 * lax.rsqrt(var + 1e-5)
        return jnp.diag(s), beta1 - mean * s
    Sn, on = fold(st_n, N_NODE)
    Se, oe = fold(st_e, N_EDGE)
    Sc, oc = fold(st_c, N_CYC)

    accs2 = layer((z_n, z_e, z_c), (HD, HD, HD),
                  (Sn, Se, Sc), (on, oe, oc), al2, ar2, fc2_w, fc2_b)
    st_n2 = _combine_tc(N_NODE, 3, 1.0 / 3.0,
                        (accs2[0], accs2[2], accs2[4]),
                        fc2_b[0] + fc2_b[2] + fc2_b[4], False)[0]
    st_e2 = _combine_tc(N_EDGE, 1, 1.0, (accs2[1],), fc2_b[1], False)[0]
    st_c2 = _combine_tc(N_CYC, 1, 1.0, (accs2[3],), fc2_b[3], False)[0]

    return _readout_tc(st_n2, st_e2, st_c2, g2, beta2, Wout, bout)
